# baseline (speedup 1.0000x reference)
import functools

import jax
import jax.numpy as jnp
from jax import lax
from jax.experimental import pallas as pl
from jax.experimental.pallas import tpu as pltpu
from jax.experimental.pallas import tpu_sc as plsc

_TILE = 256
_NTILES = 24
_CHUNK = 1024


def _prep_body(x_ref, g_ref, b_ref, wg_ref, xn_ref, pos_ref, tok_ref,
               wp_ref, te_ref, *, n, n_experts, tile, n_tiles):
    x_t = x_ref[...]
    mean = jnp.mean(x_t, axis=1, keepdims=True)
    cent = x_t - mean
    var = jnp.mean(cent * cent, axis=1, keepdims=True)
    xn = cent * lax.rsqrt(var + 1e-5)
    xn = xn * g_ref[...] + b_ref[...]
    xn_ref[...] = xn

    logits = lax.dot_general(xn, wg_ref[...], (((1,), (1,)), ((), ())),
                             preferred_element_type=jnp.float32)
    lmax = jnp.max(logits, axis=1, keepdims=True)
    ex = jnp.exp(logits - lmax)
    probs = ex / jnp.sum(ex, axis=1, keepdims=True)
    iota_e = lax.broadcasted_iota(jnp.int32, probs.shape, 1)
    v1 = jnp.max(probs, axis=1, keepdims=True)
    idx1 = jnp.min(jnp.where(probs == v1, iota_e, n_experts), axis=1,
                   keepdims=True)
    pmask = jnp.where(iota_e == idx1, -jnp.inf, probs)
    v2 = jnp.max(pmask, axis=1, keepdims=True)
    idx2 = jnp.min(jnp.where(pmask == v2, iota_e, n_experts), axis=1,
                   keepdims=True)
    denom = v1 + v2 + 1e-9
    wp_ref[...] = jnp.concatenate([v1 / denom, v2 / denom],
                                  axis=0).reshape(2 * n)

    m_all = jnp.concatenate(
        [(iota_e == idx1).astype(jnp.float32),
         (iota_e == idx2).astype(jnp.float32)], axis=0)

    c = min(_CHUNK, 2 * n)
    n_chunks = (2 * n) // c
    ri = lax.broadcasted_iota(jnp.int32, (c, c), 0)
    ci = lax.broadcasted_iota(jnp.int32, (c, c), 1)
    tril = (ci < ri).astype(jnp.float32)
    base = jnp.zeros((1, n_experts), jnp.float32)
    ranks = []
    for k in range(n_chunks):
        mc = m_all[k * c:(k + 1) * c]
        r_loc = lax.dot_general(tril, mc, (((1,), (0,)), ((), ())),
                                preferred_element_type=jnp.float32) + base
        ranks.append(jnp.sum(mc * r_loc, axis=1, keepdims=True))
        base = base + jnp.sum(mc, axis=0, keepdims=True)
    rank = jnp.concatenate(ranks, axis=0)
    counts = base

    cnt_i = counts.astype(jnp.int32)
    pc = ((cnt_i + (tile - 1)) // tile) * tile
    e_r = lax.broadcasted_iota(jnp.int32, (n_experts, n_experts), 0)
    e_c = lax.broadcasted_iota(jnp.int32, (n_experts, n_experts), 1)
    ltri8 = (e_r < e_c).astype(jnp.float32)
    off = lax.dot_general(pc.astype(jnp.float32), ltri8,
                          (((1,), (0,)), ((), ())),
                          preferred_element_type=jnp.float32)

    pos = rank + jnp.sum(m_all * off, axis=1, keepdims=True)
    pos_ref[...] = pos.astype(jnp.int32).reshape(2 * n)
    tok_iota = lax.broadcasted_iota(jnp.int32, (2 * n, 1), 0)
    tok_ref[...] = jnp.where(tok_iota >= n, tok_iota - n,
                             tok_iota).reshape(2 * n)

    end_t = ((off + pc.astype(jnp.float32)) /
             float(tile)).astype(jnp.int32)
    ti = lax.broadcasted_iota(jnp.int32, (32, 1), 0)
    te = jnp.sum((ti >= end_t).astype(jnp.int32), axis=1, keepdims=True)
    iexp = lax.broadcasted_iota(jnp.int32, (1, n_experts), 1)
    laste = jnp.max(jnp.where(pc > 0, iexp, 0), axis=1, keepdims=True)
    n_real = jnp.sum(pc, axis=1, keepdims=True) // tile
    te = jnp.where(te >= n_experts, laste, te)
    te_ref[...] = jnp.where(ti == 31, n_real, te).reshape(32)


def _ffn_body(te_ref, xs_ref, sw_ref, w1_ref, b1_ref, w2_ref, b2_ref,
              out_ref):
    i = pl.program_id(0)

    @pl.when(i < te_ref[31])
    def _():
        x_t = xs_ref[...]
        h = lax.dot_general(x_t, w1_ref[0], (((1,), (1,)), ((), ())),
                            preferred_element_type=jnp.float32) + b1_ref[0]
        h = jax.nn.gelu(h, approximate=True)
        o = lax.dot_general(h, w2_ref[0], (((1,), (1,)), ((), ())),
                            preferred_element_type=jnp.float32) + b2_ref[0]
        out_ref[...] = o * sw_ref[...]


def _make_dispatch(n_pairs, d, p_max):
    info = plsc.get_sparse_core_info()
    nc, ns = info.num_cores, info.num_subcores
    nw = nc * ns
    per_w = n_pairs // nw
    mesh = plsc.VectorSubcoreMesh(core_axis_name="c", subcore_axis_name="s")

    @functools.partial(
        pl.kernel, mesh=mesh,
        out_type=(jax.ShapeDtypeStruct((p_max, d), jnp.float32),
                  jax.ShapeDtypeStruct((p_max,), jnp.float32)),
        scratch_types=[
            pltpu.VMEM((per_w,), jnp.int32),
            pltpu.VMEM((per_w,), jnp.int32),
            pltpu.VMEM((per_w,), jnp.float32),
            pltpu.VMEM((per_w, d), jnp.float32),
            pltpu.SemaphoreType.DMA,
        ],
    )
    def dispatch(xn_hbm, tok_hbm, pos_hbm, wp_hbm, xs_hbm, sw_hbm,
                 tok_v, pos_v, w_v, rows_v, sem):
        wid = lax.axis_index("s") * nc + lax.axis_index("c")
        base = wid * per_w
        pltpu.sync_copy(tok_hbm.at[pl.ds(base, per_w)], tok_v)
        pltpu.sync_copy(pos_hbm.at[pl.ds(base, per_w)], pos_v)
        pltpu.sync_copy(wp_hbm.at[pl.ds(base, per_w)], w_v)
        pltpu.async_copy(xn_hbm.at[tok_v], rows_v, sem).wait()
        pltpu.async_copy(rows_v, xs_hbm.at[pos_v], sem).wait()
        pltpu.async_copy(w_v, sw_hbm.at[pos_v], sem).wait()

    return dispatch


def _make_collect_combine(n_tok, d, p_max):
    info = plsc.get_sparse_core_info()
    nc, ns = info.num_cores, info.num_subcores
    nw = nc * ns
    per_w = n_tok // nw
    mesh = plsc.VectorSubcoreMesh(core_axis_name="c", subcore_axis_name="s")
    n_ch = d // 16

    @functools.partial(
        pl.kernel, mesh=mesh,
        out_type=jax.ShapeDtypeStruct((n_tok, d), jnp.float32),
        scratch_types=[
            pltpu.VMEM((per_w,), jnp.int32),
            pltpu.VMEM((per_w,), jnp.int32),
            pltpu.VMEM((per_w, d), jnp.float32),
            pltpu.VMEM((per_w, d), jnp.float32),
            pltpu.SemaphoreType.DMA,
        ],
    )
    def collect_combine(os_hbm, pos_hbm, x_hbm, y_hbm, p1_v, p2_v, acc_v,
                        r_v, sem):
        wid = lax.axis_index("s") * nc + lax.axis_index("c")
        base = wid * per_w
        pltpu.sync_copy(pos_hbm.at[pl.ds(base, per_w)], p1_v)
        pltpu.sync_copy(pos_hbm.at[pl.ds(n_tok + base, per_w)], p2_v)
        pltpu.sync_copy(x_hbm.at[pl.ds(base, per_w)], acc_v)

        def add_rows(r, _):
            for j in range(n_ch):
                sl = pl.ds(j * 16, 16)
                acc_v[r, sl] = acc_v[r, sl] + r_v[r, sl]
            return 0

        pltpu.async_copy(os_hbm.at[p1_v], r_v, sem).wait()
        lax.fori_loop(0, per_w, add_rows, 0)
        pltpu.async_copy(os_hbm.at[p2_v], r_v, sem).wait()
        lax.fori_loop(0, per_w, add_rows, 0)
        pltpu.sync_copy(acc_v, y_hbm.at[pl.ds(base, per_w)])

    return collect_combine


def kernel(x, ln_gamma, ln_beta, Wg, W1, b1, W2, b2):
    B, N, D = x.shape
    E, H, _ = W1.shape
    x2 = x.reshape(N, D)
    p_max = _NTILES * _TILE

    xn, pos, tok, wp, te = pl.pallas_call(
        functools.partial(_prep_body, n=N, n_experts=E, tile=_TILE,
                          n_tiles=_NTILES),
        in_specs=[
            pl.BlockSpec((N, D), lambda: (0, 0)),
            pl.BlockSpec((1, D), lambda: (0, 0)),
            pl.BlockSpec((1, D), lambda: (0, 0)),
            pl.BlockSpec((E, D), lambda: (0, 0)),
        ],
        out_specs=(
            pl.BlockSpec((N, D), lambda: (0, 0)),
            pl.BlockSpec((2 * N,), lambda: (0,)),
            pl.BlockSpec((2 * N,), lambda: (0,)),
            pl.BlockSpec((2 * N,), lambda: (0,)),
            pl.BlockSpec((32,), lambda: (0,)),
        ),
        out_shape=(
            jax.ShapeDtypeStruct((N, D), jnp.float32),
            jax.ShapeDtypeStruct((2 * N,), jnp.int32),
            jax.ShapeDtypeStruct((2 * N,), jnp.int32),
            jax.ShapeDtypeStruct((2 * N,), jnp.float32),
            jax.ShapeDtypeStruct((32,), jnp.int32),
        ),
    )(x2, ln_gamma.reshape(1, D), ln_beta.reshape(1, D), Wg)

    xs, sw = _make_dispatch(2 * N, D, p_max)(xn, tok, pos, wp)

    os_ = pl.pallas_call(
        _ffn_body,
        grid_spec=pltpu.PrefetchScalarGridSpec(
            num_scalar_prefetch=1,
            grid=(_NTILES,),
            in_specs=[
                pl.BlockSpec((_TILE, D), lambda i, te_r: (i, 0)),
                pl.BlockSpec((_TILE, 1), lambda i, te_r: (i, 0)),
                pl.BlockSpec((1, H, D), lambda i, te_r: (te_r[i], 0, 0)),
                pl.BlockSpec((1, 1, H), lambda i, te_r: (te_r[i], 0, 0)),
                pl.BlockSpec((1, D, H), lambda i, te_r: (te_r[i], 0, 0)),
                pl.BlockSpec((1, 1, D), lambda i, te_r: (te_r[i], 0, 0)),
            ],
            out_specs=pl.BlockSpec((_TILE, D), lambda i, te_r: (i, 0)),
        ),
        out_shape=jax.ShapeDtypeStruct((p_max, D), jnp.float32),
    )(te, xs, sw.reshape(p_max, 1), W1, b1.reshape(E, 1, H), W2,
      b2.reshape(E, 1, D))

    out = _make_collect_combine(N, D, p_max)(os_, pos, x2)
    return out.reshape(B, N, D)


# speedup vs baseline: 1.1972x; 1.1972x over previous
import functools

import jax
import jax.numpy as jnp
from jax import lax
from jax.experimental import pallas as pl
from jax.experimental.pallas import tpu as pltpu
from jax.experimental.pallas import tpu_sc as plsc

_TILE = 256
_NTILES = 24
_CHUNK = 1024


def _prep_body(x_ref, g_ref, b_ref, wg_ref, xn_ref, pos_ref, tok_ref,
               wp_ref, te_ref, *, n, n_experts, tile, n_tiles):
    x_t = x_ref[...]
    mean = jnp.mean(x_t, axis=1, keepdims=True)
    cent = x_t - mean
    var = jnp.mean(cent * cent, axis=1, keepdims=True)
    xn = cent * lax.rsqrt(var + 1e-5)
    xn = xn * g_ref[...] + b_ref[...]
    xn_ref[...] = xn

    logits = lax.dot_general(xn, wg_ref[...], (((1,), (1,)), ((), ())),
                             preferred_element_type=jnp.float32)
    lmax = jnp.max(logits, axis=1, keepdims=True)
    ex = jnp.exp(logits - lmax)
    probs = ex / jnp.sum(ex, axis=1, keepdims=True)
    iota_e = lax.broadcasted_iota(jnp.int32, probs.shape, 1)
    v1 = jnp.max(probs, axis=1, keepdims=True)
    idx1 = jnp.min(jnp.where(probs == v1, iota_e, n_experts), axis=1,
                   keepdims=True)
    pmask = jnp.where(iota_e == idx1, -jnp.inf, probs)
    v2 = jnp.max(pmask, axis=1, keepdims=True)
    idx2 = jnp.min(jnp.where(pmask == v2, iota_e, n_experts), axis=1,
                   keepdims=True)
    denom = v1 + v2 + 1e-9
    wp_ref[...] = jnp.broadcast_to(
        jnp.concatenate([v1 / denom, v2 / denom], axis=0), (2 * n, 128))

    m_all = jnp.concatenate(
        [(iota_e == idx1).astype(jnp.float32),
         (iota_e == idx2).astype(jnp.float32)], axis=0)

    c = min(_CHUNK, 2 * n)
    n_chunks = (2 * n) // c
    ri = lax.broadcasted_iota(jnp.int32, (c, c), 0)
    ci = lax.broadcasted_iota(jnp.int32, (c, c), 1)
    tril = (ci < ri).astype(jnp.float32)
    base = jnp.zeros((1, n_experts), jnp.float32)
    ranks = []
    for k in range(n_chunks):
        mc = m_all[k * c:(k + 1) * c]
        r_loc = lax.dot_general(tril, mc, (((1,), (0,)), ((), ())),
                                preferred_element_type=jnp.float32) + base
        ranks.append(jnp.sum(mc * r_loc, axis=1, keepdims=True))
        base = base + jnp.sum(mc, axis=0, keepdims=True)
    rank = jnp.concatenate(ranks, axis=0)
    counts = base

    cnt_i = counts.astype(jnp.int32)
    pc = ((cnt_i + (tile - 1)) // tile) * tile
    e_r = lax.broadcasted_iota(jnp.int32, (n_experts, n_experts), 0)
    e_c = lax.broadcasted_iota(jnp.int32, (n_experts, n_experts), 1)
    ltri8 = (e_r < e_c).astype(jnp.float32)
    off = lax.dot_general(pc.astype(jnp.float32), ltri8,
                          (((1,), (0,)), ((), ())),
                          preferred_element_type=jnp.float32)

    pos = rank + jnp.sum(m_all * off, axis=1, keepdims=True)
    pos_ref[...] = pos.astype(jnp.int32).reshape(2 * n)
    tok_iota = lax.broadcasted_iota(jnp.int32, (2 * n, 1), 0)
    tok_ref[...] = jnp.where(tok_iota >= n, tok_iota - n,
                             tok_iota).reshape(2 * n)

    end_t = ((off + pc.astype(jnp.float32)) /
             float(tile)).astype(jnp.int32)
    ti = lax.broadcasted_iota(jnp.int32, (32, 1), 0)
    te = jnp.sum((ti >= end_t).astype(jnp.int32), axis=1, keepdims=True)
    iexp = lax.broadcasted_iota(jnp.int32, (1, n_experts), 1)
    laste = jnp.max(jnp.where(pc > 0, iexp, 0), axis=1, keepdims=True)
    n_real = jnp.sum(pc, axis=1, keepdims=True) // tile
    te = jnp.where(te >= n_experts, laste, te)
    te_ref[...] = jnp.where(ti == 31, n_real, te).reshape(32)


def _ffn_body(te_ref, xs_ref, sw_ref, w1_ref, b1_ref, w2_ref, b2_ref,
              out_ref):
    i = pl.program_id(0)

    @pl.when(i < te_ref[31])
    def _():
        x_t = xs_ref[...]
        h = lax.dot_general(x_t, w1_ref[0], (((1,), (1,)), ((), ())),
                            preferred_element_type=jnp.float32) + b1_ref[0]
        h = jax.nn.gelu(h, approximate=True)
        o = lax.dot_general(h, w2_ref[0], (((1,), (1,)), ((), ())),
                            preferred_element_type=jnp.float32) + b2_ref[0]
        out_ref[...] = o * sw_ref[:, 0:1]


def _make_dispatch(n_pairs, d, p_max):
    info = plsc.get_sparse_core_info()
    nc, ns = info.num_cores, info.num_subcores
    nw = nc * ns
    per_w = n_pairs // nw
    mesh = plsc.VectorSubcoreMesh(core_axis_name="c", subcore_axis_name="s")

    @functools.partial(
        pl.kernel, mesh=mesh,
        out_type=(jax.ShapeDtypeStruct((p_max, d), jnp.float32),
                  jax.ShapeDtypeStruct((p_max, 128), jnp.float32)),
        scratch_types=[
            pltpu.VMEM((per_w,), jnp.int32),
            pltpu.VMEM((per_w,), jnp.int32),
            pltpu.VMEM((per_w, 128), jnp.float32),
            pltpu.VMEM((per_w, d), jnp.float32),
            pltpu.SemaphoreType.DMA,
        ],
    )
    def dispatch(xn_hbm, tok_hbm, pos_hbm, wp_hbm, xs_hbm, sw_hbm,
                 tok_v, pos_v, w_v, rows_v, sem):
        wid = lax.axis_index("s") * nc + lax.axis_index("c")
        base = wid * per_w
        pltpu.sync_copy(tok_hbm.at[pl.ds(base, per_w)], tok_v)
        pltpu.sync_copy(pos_hbm.at[pl.ds(base, per_w)], pos_v)
        pltpu.sync_copy(wp_hbm.at[pl.ds(base, per_w)], w_v)
        wsc = pltpu.async_copy(w_v, sw_hbm.at[pos_v], sem)
        pltpu.async_copy(xn_hbm.at[tok_v], rows_v, sem).wait()
        pltpu.async_copy(rows_v, xs_hbm.at[pos_v], sem).wait()
        wsc.wait()

    return dispatch


def _make_collect_combine(n_tok, d, p_max):
    info = plsc.get_sparse_core_info()
    nc, ns = info.num_cores, info.num_subcores
    nw = nc * ns
    per_w = n_tok // nw
    mesh = plsc.VectorSubcoreMesh(core_axis_name="c", subcore_axis_name="s")
    n_ch = d // 16

    @functools.partial(
        pl.kernel, mesh=mesh,
        out_type=jax.ShapeDtypeStruct((n_tok, d), jnp.float32),
        scratch_types=[
            pltpu.VMEM((per_w,), jnp.int32),
            pltpu.VMEM((per_w,), jnp.int32),
            pltpu.VMEM((per_w, d), jnp.float32),
            pltpu.VMEM((per_w, d), jnp.float32),
            pltpu.SemaphoreType.DMA,
        ],
    )
    def collect_combine(os_hbm, pos_hbm, x_hbm, y_hbm, p1_v, p2_v, acc_v,
                        r_v, sem):
        wid = lax.axis_index("s") * nc + lax.axis_index("c")
        base = wid * per_w
        pltpu.sync_copy(pos_hbm.at[pl.ds(base, per_w)], p1_v)
        pltpu.sync_copy(pos_hbm.at[pl.ds(n_tok + base, per_w)], p2_v)
        pltpu.sync_copy(x_hbm.at[pl.ds(base, per_w)], acc_v)

        def add_rows(r, _):
            for j in range(n_ch):
                sl = pl.ds(j * 16, 16)
                acc_v[r, sl] = acc_v[r, sl] + r_v[r, sl]
            return 0

        pltpu.async_copy(os_hbm.at[p1_v], r_v, sem).wait()
        lax.fori_loop(0, per_w, add_rows, 0)
        pltpu.async_copy(os_hbm.at[p2_v], r_v, sem).wait()
        lax.fori_loop(0, per_w, add_rows, 0)
        pltpu.sync_copy(acc_v, y_hbm.at[pl.ds(base, per_w)])

    return collect_combine


def kernel(x, ln_gamma, ln_beta, Wg, W1, b1, W2, b2):
    B, N, D = x.shape
    E, H, _ = W1.shape
    x2 = x.reshape(N, D)
    p_max = _NTILES * _TILE

    xn, pos, tok, wp, te = pl.pallas_call(
        functools.partial(_prep_body, n=N, n_experts=E, tile=_TILE,
                          n_tiles=_NTILES),
        in_specs=[
            pl.BlockSpec((N, D), lambda: (0, 0)),
            pl.BlockSpec((1, D), lambda: (0, 0)),
            pl.BlockSpec((1, D), lambda: (0, 0)),
            pl.BlockSpec((E, D), lambda: (0, 0)),
        ],
        out_specs=(
            pl.BlockSpec((N, D), lambda: (0, 0)),
            pl.BlockSpec((2 * N,), lambda: (0,)),
            pl.BlockSpec((2 * N,), lambda: (0,)),
            pl.BlockSpec((2 * N, 128), lambda: (0, 0)),
            pl.BlockSpec((32,), lambda: (0,)),
        ),
        out_shape=(
            jax.ShapeDtypeStruct((N, D), jnp.float32),
            jax.ShapeDtypeStruct((2 * N,), jnp.int32),
            jax.ShapeDtypeStruct((2 * N,), jnp.int32),
            jax.ShapeDtypeStruct((2 * N, 128), jnp.float32),
            jax.ShapeDtypeStruct((32,), jnp.int32),
        ),
    )(x2, ln_gamma.reshape(1, D), ln_beta.reshape(1, D), Wg)

    xs, sw = _make_dispatch(2 * N, D, p_max)(xn, tok, pos, wp)

    os_ = pl.pallas_call(
        _ffn_body,
        grid_spec=pltpu.PrefetchScalarGridSpec(
            num_scalar_prefetch=1,
            grid=(_NTILES,),
            in_specs=[
                pl.BlockSpec((_TILE, D), lambda i, te_r: (i, 0)),
                pl.BlockSpec((_TILE, 128), lambda i, te_r: (i, 0)),
                pl.BlockSpec((1, H, D), lambda i, te_r: (te_r[i], 0, 0)),
                pl.BlockSpec((1, 1, H), lambda i, te_r: (te_r[i], 0, 0)),
                pl.BlockSpec((1, D, H), lambda i, te_r: (te_r[i], 0, 0)),
                pl.BlockSpec((1, 1, D), lambda i, te_r: (te_r[i], 0, 0)),
            ],
            out_specs=pl.BlockSpec((_TILE, D), lambda i, te_r: (i, 0)),
        ),
        out_shape=jax.ShapeDtypeStruct((p_max, D), jnp.float32),
    )(te, xs, sw, W1, b1.reshape(E, 1, H), W2, b2.reshape(E, 1, D))

    out = _make_collect_combine(N, D, p_max)(os_, pos, x2)
    return out.reshape(B, N, D)
